# baseline (speedup 1.0000x reference)
import functools

import jax
import jax.numpy as jnp
from jax import lax
from jax.experimental import pallas as pl
from jax.experimental.pallas import tpu as pltpu
from jax.experimental.pallas import tpu_sc as plsc

_NUM_SEGMENTS = 10000
_SEG_HALF = 5120
_SEG_PAD = 2 * _SEG_HALF
_ACC_ROWS = _SEG_HALF + 8
_N_TOKENS = 320000
_D = 128
_SUB = 128
_BLOCK = 256
_NSUB = _BLOCK // _SUB
_NBLOCKS = _N_TOKENS // _BLOCK
_NC = 2
_NS = 16
_RPT = _SEG_HALF // _NS
_HPT = _RPT // 2


_mesh = plsc.VectorSubcoreMesh(core_axis_name="c", subcore_axis_name="s")


@functools.partial(
    pl.kernel,
    mesh=_mesh,
    out_type=jax.ShapeDtypeStruct((_SEG_PAD, _D), jnp.float32),
    scratch_types=[
        pltpu.VMEM((2, _NSUB, _SUB), jnp.int32),
        pltpu.VMEM((2, _NSUB, _SUB), jnp.int32),
        pltpu.VMEM((2, _BLOCK, _D), jnp.float32),
        pltpu.VMEM((_SUB,), jnp.float32),
        pltpu.VMEM((32, _D), jnp.float32),
        pltpu.VMEM((_RPT,), jnp.float32),
        pltpu.VMEM((16,), jnp.int32),
        pltpu.VMEM_SHARED((_ACC_ROWS, _D), jnp.float32),
        pltpu.VMEM_SHARED((_SEG_PAD,), jnp.float32),
        pltpu.SemaphoreType.DMA((2,)),
        pltpu.SemaphoreType.DMA((2,)),
        pltpu.SemaphoreType.DMA((2,)),
        pltpu.SemaphoreType.DMA((2,)),
    ],
)
def _sc_mean(enc_hbm, ids_hbm, tlo_hbm, out_hbm,
             idx_v, idx2_v, rows_v, ones_v, zero_v, cnt_v, tlo_v, acc_sh, cnt_sh,
             sem_rows, sem_ids, sem_w, sem_c):
    cid = lax.axis_index("c")
    sid = lax.axis_index("s")

    pltpu.sync_copy(tlo_hbm, tlo_v)

    for j in range(_SUB // 16):
        ones_v[pl.ds(j * 16, 16)] = jnp.ones((16,), jnp.float32)

    def zrow(r, carry):
        for j in range(_D // 16):
            zero_v[r, pl.ds(j * 16, 16)] = jnp.zeros((16,), jnp.float32)
        return carry

    lax.fori_loop(0, 32, zrow, 0)

    base_row = sid * _RPT

    def zacc(t, carry):
        pltpu.sync_copy(zero_v, acc_sh.at[pl.ds(base_row + t * 32, 32)])
        return carry

    lax.fori_loop(0, _RPT // 32, zacc, 0)

    cnt_base = sid * (_SEG_PAD // _NS)

    def zcnt(t, carry):
        pltpu.sync_copy(zero_v.at[0], cnt_sh.at[pl.ds(cnt_base + t * 128, 128)])
        return carry

    lax.fori_loop(0, (_SEG_PAD // _NS) // 128, zcnt, 0)

    plsc.subcore_barrier()

    t_lo = tlo_v[...][0]
    lo = jnp.where(cid == 0, 0, t_lo // _BLOCK)
    hi = jnp.where(cid == 0, (t_lo + _BLOCK - 1) // _BLOCK, _NBLOCKS)
    n_c = hi - lo
    per = n_c // _NS
    rem = n_c - per * _NS
    base = lo + sid * per + jnp.minimum(sid, rem)
    n_my = per + jnp.where(sid < rem, 1, 0)
    seg_base = cid * _SEG_HALF

    def _start_load(c, b):
        pltpu.async_copy(enc_hbm.at[pl.ds(c * _BLOCK, _BLOCK)], rows_v.at[b],
                         sem_rows.at[b])
        pltpu.async_copy(ids_hbm.at[c], idx_v.at[b], sem_ids.at[b])

    def _wait_load(c, b):
        pltpu.make_async_copy(enc_hbm.at[pl.ds(c * _BLOCK, _BLOCK)],
                              rows_v.at[b], sem_rows.at[b]).wait()
        pltpu.make_async_copy(ids_hbm.at[c], idx_v.at[b],
                              sem_ids.at[b]).wait()

    @pl.when(n_my > 0)
    def _prime():
        _start_load(base, 0)

    def body(i, carry):
        b = i % 2
        bn = (i + 1) % 2

        @pl.when(i >= 1)
        def _drain():
            for j in range(_NSUB):
                pltpu.make_async_copy(ones_v, cnt_sh.at[idx_v.at[bn, j]],
                                      sem_c.at[bn]).wait()

        @pl.when(i + 1 < n_my)
        def _next():
            _start_load(base + i + 1, bn)

        _wait_load(base + i, b)
        for j in range(_NSUB):
            pltpu.async_copy(ones_v, cnt_sh.at[idx_v.at[b, j]],
                             sem_c.at[b], add=True)
            for k in range(_SUB // 16):
                v = idx_v[b, j, pl.ds(k * 16, 16)] - seg_base
                oob = (v < 0) | (v >= _SEG_HALF)
                idx2_v[b, j, pl.ds(k * 16, 16)] = jnp.where(oob, _SEG_HALF, v)
            pltpu.sync_copy(rows_v.at[b, pl.ds(j * _SUB, _SUB)],
                            acc_sh.at[idx2_v.at[b, j]], add=True)
        return carry

    lax.fori_loop(0, n_my, body, 0)

    @pl.when(n_my > 0)
    def _drain_last():
        blast = (n_my - 1) % 2
        for j in range(_NSUB):
            pltpu.make_async_copy(ones_v, cnt_sh.at[idx_v.at[blast, j]],
                                  sem_c.at[blast]).wait()

    plsc.subcore_barrier()

    pltpu.sync_copy(cnt_sh.at[pl.ds(seg_base + base_row, _RPT)], cnt_v)

    def recip(k, carry):
        cv = cnt_v[pl.ds(k * 16, 16)]
        cnt_v[pl.ds(k * 16, 16)] = 1.0 / jnp.maximum(cv, 1.0)
        return carry

    lax.fori_loop(0, _RPT // 16, recip, 0)

    seg0 = seg_base + base_row
    for p in range(2):
        pltpu.sync_copy(acc_sh.at[pl.ds(base_row + p * _HPT, _HPT)],
                        rows_v.at[p, pl.ds(0, _HPT)])

        def divgrp(g, carry):
            m16 = cnt_v[pl.ds(p * _HPT + g * 16, 16)]
            for rr in range(16):
                r = g * 16 + rr
                m = lax.broadcast(m16[rr], (16,))
                for k in range(_D // 16):
                    rows_v[p, r, pl.ds(k * 16, 16)] = (
                        rows_v[p, r, pl.ds(k * 16, 16)] * m)
            return carry

        lax.fori_loop(0, _HPT // 16, divgrp, 0)
        pltpu.async_copy(rows_v.at[p, pl.ds(0, _HPT)],
                         out_hbm.at[pl.ds(seg0 + p * _HPT, _HPT)],
                         sem_w.at[p])

    for p in range(2):
        pltpu.make_async_copy(rows_v.at[p, pl.ds(0, _HPT)],
                              out_hbm.at[pl.ds(seg0 + p * _HPT, _HPT)],
                              sem_w.at[p]).wait()


@jax.jit
def _impl(enc_seq, segment_ids):
    ids3d = segment_ids.reshape(_NBLOCKS, _NSUB, _SUB)
    t_lo = jnp.sum((segment_ids < _SEG_HALF).astype(jnp.int32)).astype(jnp.int32)
    tlo16 = jnp.broadcast_to(t_lo, (16,))
    padded = _sc_mean(enc_seq, ids3d, tlo16)
    return padded[:_NUM_SEGMENTS]


def kernel(enc_seq, segment_ids):
    return _impl(enc_seq, segment_ids)


# speedup vs baseline: 1.0129x; 1.0129x over previous
import functools

import jax
import jax.numpy as jnp
from jax import lax
from jax.experimental import pallas as pl
from jax.experimental.pallas import tpu as pltpu
from jax.experimental.pallas import tpu_sc as plsc

_NUM_SEGMENTS = 10000
_SEG_HALF = 5120
_SEG_PAD = 2 * _SEG_HALF
_ACC_ROWS = _SEG_HALF + 8
_N_TOKENS = 320000
_D = 128
_SUB = 128
_BLOCK = 256
_NSUB = _BLOCK // _SUB
_NBLOCKS = _N_TOKENS // _BLOCK
_NC = 2
_NS = 16
_RPT = _SEG_HALF // _NS
_HPT = _RPT // 2


_mesh = plsc.VectorSubcoreMesh(core_axis_name="c", subcore_axis_name="s")


@functools.partial(
    pl.kernel,
    mesh=_mesh,
    out_type=jax.ShapeDtypeStruct((_SEG_PAD, _D), jnp.float32),
    scratch_types=[
        pltpu.VMEM((2, _NSUB, _SUB), jnp.int32),
        pltpu.VMEM((2, _NSUB, _SUB), jnp.int32),
        pltpu.VMEM((2, _BLOCK, _D), jnp.float32),
        pltpu.VMEM((_SUB,), jnp.float32),
        pltpu.VMEM((32, _D), jnp.float32),
        pltpu.VMEM((_RPT,), jnp.float32),
        pltpu.VMEM((16,), jnp.int32),
        pltpu.VMEM_SHARED((_ACC_ROWS, _D), jnp.float32),
        pltpu.VMEM_SHARED((_SEG_PAD,), jnp.float32),
        pltpu.SemaphoreType.DMA((2,)),
        pltpu.SemaphoreType.DMA((2,)),
        pltpu.SemaphoreType.DMA((2,)),
        pltpu.SemaphoreType.DMA((2,)),
    ],
)
def _sc_mean(enc_hbm, ids_hbm, tlo_hbm, out_hbm,
             idx_v, idx2_v, rows_v, ones_v, zero_v, cnt_v, tlo_v, acc_sh, cnt_sh,
             sem_rows, sem_ids, sem_w, sem_c):
    cid = lax.axis_index("c")
    sid = lax.axis_index("s")

    pltpu.sync_copy(tlo_hbm, tlo_v)

    for j in range(_SUB // 16):
        ones_v[pl.ds(j * 16, 16)] = jnp.ones((16,), jnp.float32)

    def zrow(r, carry):
        for j in range(_D // 16):
            zero_v[r, pl.ds(j * 16, 16)] = jnp.zeros((16,), jnp.float32)
        return carry

    lax.fori_loop(0, 32, zrow, 0)

    base_row = sid * _RPT

    t_lo = tlo_v[...][0]
    lo = jnp.where(cid == 0, 0, t_lo // _BLOCK)
    hi = jnp.where(cid == 0, (t_lo + _BLOCK - 1) // _BLOCK, _NBLOCKS)
    n_c = hi - lo
    per = n_c // _NS
    rem = n_c - per * _NS
    base = lo + sid * per + jnp.minimum(sid, rem)
    n_my = per + jnp.where(sid < rem, 1, 0)
    seg_base = cid * _SEG_HALF

    def _start_load(c, b):
        pltpu.async_copy(enc_hbm.at[pl.ds(c * _BLOCK, _BLOCK)], rows_v.at[b],
                         sem_rows.at[b])
        pltpu.async_copy(ids_hbm.at[c], idx_v.at[b], sem_ids.at[b])

    def _wait_load(c, b):
        pltpu.make_async_copy(enc_hbm.at[pl.ds(c * _BLOCK, _BLOCK)],
                              rows_v.at[b], sem_rows.at[b]).wait()
        pltpu.make_async_copy(ids_hbm.at[c], idx_v.at[b],
                              sem_ids.at[b]).wait()

    @pl.when(n_my > 0)
    def _prime():
        _start_load(base, 0)

    @pl.when(n_my > 1)
    def _prime2():
        _start_load(base + 1, 1)

    cnt_base = sid * (_SEG_PAD // _NS)

    def zacc(t, carry):
        pltpu.async_copy(zero_v, acc_sh.at[pl.ds(base_row + t * 32, 32)],
                         sem_w.at[0])
        return carry

    lax.fori_loop(0, _RPT // 32, zacc, 0)

    def zcnt(t, carry):
        pltpu.async_copy(zero_v.at[0], cnt_sh.at[pl.ds(cnt_base + t * 128, 128)],
                         sem_w.at[1])
        return carry

    lax.fori_loop(0, (_SEG_PAD // _NS) // 128, zcnt, 0)

    def zacc_d(t, carry):
        pltpu.make_async_copy(zero_v, acc_sh.at[pl.ds(base_row + t * 32, 32)],
                              sem_w.at[0]).wait()
        return carry

    lax.fori_loop(0, _RPT // 32, zacc_d, 0)

    def zcnt_d(t, carry):
        pltpu.make_async_copy(zero_v.at[0],
                              cnt_sh.at[pl.ds(cnt_base + t * 128, 128)],
                              sem_w.at[1]).wait()
        return carry

    lax.fori_loop(0, (_SEG_PAD // _NS) // 128, zcnt_d, 0)

    plsc.subcore_barrier()

    def body(i, carry):
        b = i % 2

        _wait_load(base + i, b)
        for j in range(_NSUB):
            pltpu.async_copy(ones_v, cnt_sh.at[idx_v.at[b, j]],
                             sem_c.at[b], add=True)
            for k in range(_SUB // 16):
                v = idx_v[b, j, pl.ds(k * 16, 16)] - seg_base
                oob = (v < 0) | (v >= _SEG_HALF)
                idx2_v[b, j, pl.ds(k * 16, 16)] = jnp.where(oob, _SEG_HALF, v)
            pltpu.sync_copy(rows_v.at[b, pl.ds(j * _SUB, _SUB)],
                            acc_sh.at[idx2_v.at[b, j]], add=True)
        for j in range(_NSUB):
            pltpu.make_async_copy(ones_v, cnt_sh.at[idx_v.at[b, j]],
                                  sem_c.at[b]).wait()

        @pl.when(i + 2 < n_my)
        def _next():
            _start_load(base + i + 2, b)

        return carry

    lax.fori_loop(0, n_my, body, 0)

    plsc.subcore_barrier()

    pltpu.sync_copy(cnt_sh.at[pl.ds(seg_base + base_row, _RPT)], cnt_v)

    def recip(k, carry):
        cv = cnt_v[pl.ds(k * 16, 16)]
        cnt_v[pl.ds(k * 16, 16)] = 1.0 / jnp.maximum(cv, 1.0)
        return carry

    lax.fori_loop(0, _RPT // 16, recip, 0)

    seg0 = seg_base + base_row
    for p in range(2):
        pltpu.sync_copy(acc_sh.at[pl.ds(base_row + p * _HPT, _HPT)],
                        rows_v.at[p, pl.ds(0, _HPT)])

        def divgrp(g, carry):
            m16 = cnt_v[pl.ds(p * _HPT + g * 16, 16)]
            for rr in range(16):
                r = g * 16 + rr
                m = lax.broadcast(m16[rr], (16,))
                for k in range(_D // 16):
                    rows_v[p, r, pl.ds(k * 16, 16)] = (
                        rows_v[p, r, pl.ds(k * 16, 16)] * m)
            return carry

        lax.fori_loop(0, _HPT // 16, divgrp, 0)
        pltpu.async_copy(rows_v.at[p, pl.ds(0, _HPT)],
                         out_hbm.at[pl.ds(seg0 + p * _HPT, _HPT)],
                         sem_w.at[p])

    for p in range(2):
        pltpu.make_async_copy(rows_v.at[p, pl.ds(0, _HPT)],
                              out_hbm.at[pl.ds(seg0 + p * _HPT, _HPT)],
                              sem_w.at[p]).wait()


@jax.jit
def _impl(enc_seq, segment_ids):
    ids3d = segment_ids.reshape(_NBLOCKS, _NSUB, _SUB)
    t_lo = jnp.sum((segment_ids < _SEG_HALF).astype(jnp.int32)).astype(jnp.int32)
    tlo16 = jnp.broadcast_to(t_lo, (16,))
    padded = _sc_mean(enc_seq, ids3d, tlo16)
    return padded[:_NUM_SEGMENTS]


def kernel(enc_seq, segment_ids):
    return _impl(enc_seq, segment_ids)


# speedup vs baseline: 1.0442x; 1.0309x over previous
import functools

import jax
import jax.numpy as jnp
from jax import lax
from jax.experimental import pallas as pl
from jax.experimental.pallas import tpu as pltpu
from jax.experimental.pallas import tpu_sc as plsc

_NUM_SEGMENTS = 10000
_SEG_HALF = 5120
_SEG_PAD = 2 * _SEG_HALF
_ACC_ROWS = _SEG_HALF + 8
_N_TOKENS = 320000
_D = 128
_SUB = 128
_BLOCK = 256
_NSUB = _BLOCK // _SUB
_NBLOCKS = _N_TOKENS // _BLOCK
_NC = 2
_NS = 16
_RPT = _SEG_HALF // _NS
_HPT = _RPT // 2


_mesh = plsc.VectorSubcoreMesh(core_axis_name="c", subcore_axis_name="s")


@functools.partial(
    pl.kernel,
    mesh=_mesh,
    out_type=jax.ShapeDtypeStruct((_NUM_SEGMENTS, _D), jnp.float32),
    scratch_types=[
        pltpu.VMEM((2, _NSUB, _SUB), jnp.int32),
        pltpu.VMEM((2, _NSUB, _SUB), jnp.int32),
        pltpu.VMEM((2, _BLOCK, _D), jnp.float32),
        pltpu.VMEM((_SUB,), jnp.float32),
        pltpu.VMEM((32, _D), jnp.float32),
        pltpu.VMEM((_RPT,), jnp.float32),
        pltpu.VMEM((16,), jnp.int32),
        pltpu.VMEM_SHARED((_ACC_ROWS, _D), jnp.float32),
        pltpu.VMEM_SHARED((_SEG_PAD,), jnp.float32),
        pltpu.SemaphoreType.DMA((2,)),
        pltpu.SemaphoreType.DMA((2,)),
        pltpu.SemaphoreType.DMA((2,)),
        pltpu.SemaphoreType.DMA((2,)),
    ],
)
def _sc_mean(enc_hbm, ids_hbm, tlo_hbm, out_hbm,
             idx_v, idx2_v, rows_v, ones_v, zero_v, cnt_v, tlo_v, acc_sh, cnt_sh,
             sem_rows, sem_ids, sem_w, sem_c):
    cid = lax.axis_index("c")
    sid = lax.axis_index("s")

    pltpu.sync_copy(tlo_hbm, tlo_v)

    for j in range(_SUB // 16):
        ones_v[pl.ds(j * 16, 16)] = jnp.ones((16,), jnp.float32)

    def zrow(r, carry):
        for j in range(_D // 16):
            zero_v[r, pl.ds(j * 16, 16)] = jnp.zeros((16,), jnp.float32)
        return carry

    lax.fori_loop(0, 32, zrow, 0)

    base_row = sid * _RPT

    t_lo = tlo_v[...][0]
    lo = jnp.where(cid == 0, 0, t_lo // _BLOCK)
    hi = jnp.where(cid == 0, (t_lo + _BLOCK - 1) // _BLOCK, _NBLOCKS)
    n_c = hi - lo
    per = n_c // _NS
    rem = n_c - per * _NS
    base = lo + sid * per + jnp.minimum(sid, rem)
    n_my = per + jnp.where(sid < rem, 1, 0)
    seg_base = cid * _SEG_HALF

    def _start_load(c, b):
        pltpu.async_copy(enc_hbm.at[pl.ds(c * _BLOCK, _BLOCK)], rows_v.at[b],
                         sem_rows.at[b])
        pltpu.async_copy(ids_hbm.at[c], idx_v.at[b], sem_ids.at[b])

    def _wait_load(c, b):
        pltpu.make_async_copy(enc_hbm.at[pl.ds(c * _BLOCK, _BLOCK)],
                              rows_v.at[b], sem_rows.at[b]).wait()
        pltpu.make_async_copy(ids_hbm.at[c], idx_v.at[b],
                              sem_ids.at[b]).wait()

    @pl.when(n_my > 0)
    def _prime():
        _start_load(base, 0)

    @pl.when(n_my > 1)
    def _prime2():
        _start_load(base + 1, 1)

    cnt_base = sid * (_SEG_PAD // _NS)

    def zacc(t, carry):
        pltpu.async_copy(zero_v, acc_sh.at[pl.ds(base_row + t * 32, 32)],
                         sem_w.at[0])
        return carry

    lax.fori_loop(0, _RPT // 32, zacc, 0)

    def zcnt(t, carry):
        pltpu.async_copy(zero_v.at[0], cnt_sh.at[pl.ds(cnt_base + t * 128, 128)],
                         sem_w.at[1])
        return carry

    lax.fori_loop(0, (_SEG_PAD // _NS) // 128, zcnt, 0)

    def zacc_d(t, carry):
        pltpu.make_async_copy(zero_v, acc_sh.at[pl.ds(base_row + t * 32, 32)],
                              sem_w.at[0]).wait()
        return carry

    lax.fori_loop(0, _RPT // 32, zacc_d, 0)

    def zcnt_d(t, carry):
        pltpu.make_async_copy(zero_v.at[0],
                              cnt_sh.at[pl.ds(cnt_base + t * 128, 128)],
                              sem_w.at[1]).wait()
        return carry

    lax.fori_loop(0, (_SEG_PAD // _NS) // 128, zcnt_d, 0)

    plsc.subcore_barrier()

    def body(i, carry):
        b = i % 2

        _wait_load(base + i, b)
        for j in range(_NSUB):
            pltpu.async_copy(ones_v, cnt_sh.at[idx_v.at[b, j]],
                             sem_c.at[b], add=True)
            for k in range(_SUB // 16):
                v = idx_v[b, j, pl.ds(k * 16, 16)] - seg_base
                oob = (v < 0) | (v >= _SEG_HALF)
                idx2_v[b, j, pl.ds(k * 16, 16)] = jnp.where(oob, _SEG_HALF, v)
            pltpu.sync_copy(rows_v.at[b, pl.ds(j * _SUB, _SUB)],
                            acc_sh.at[idx2_v.at[b, j]], add=True)
        for j in range(_NSUB):
            pltpu.make_async_copy(ones_v, cnt_sh.at[idx_v.at[b, j]],
                                  sem_c.at[b]).wait()

        @pl.when(i + 2 < n_my)
        def _next():
            _start_load(base + i + 2, b)

        return carry

    lax.fori_loop(0, n_my, body, 0)

    plsc.subcore_barrier()

    pltpu.sync_copy(cnt_sh.at[pl.ds(seg_base + base_row, _RPT)], cnt_v)

    def recip(k, carry):
        cv = cnt_v[pl.ds(k * 16, 16)]
        cnt_v[pl.ds(k * 16, 16)] = 1.0 / jnp.maximum(cv, 1.0)
        return carry

    lax.fori_loop(0, _RPT // 16, recip, 0)

    seg0 = seg_base + base_row
    for p in range(2):
        pltpu.sync_copy(acc_sh.at[pl.ds(base_row + p * _HPT, _HPT)],
                        rows_v.at[p, pl.ds(0, _HPT)])

        def divgrp(g, carry):
            m16 = cnt_v[pl.ds(p * _HPT + g * 16, 16)]
            for rr in range(16):
                r = g * 16 + rr
                m = lax.broadcast(m16[rr], (16,))
                for k in range(_D // 16):
                    rows_v[p, r, pl.ds(k * 16, 16)] = (
                        rows_v[p, r, pl.ds(k * 16, 16)] * m)
            return carry

        lax.fori_loop(0, _HPT // 16, divgrp, 0)
        fits = seg0 + p * _HPT + _HPT <= _NUM_SEGMENTS

        @pl.when(fits)
        def _bulk():
            pltpu.async_copy(rows_v.at[p, pl.ds(0, _HPT)],
                             out_hbm.at[pl.ds(seg0 + p * _HPT, _HPT)],
                             sem_w.at[p])

        n16 = jnp.clip(_NUM_SEGMENTS - (seg0 + p * _HPT), 0, _HPT) // 16

        @pl.when(jnp.logical_not(fits))
        def _tail():
            def wout(t, carry):
                pltpu.sync_copy(
                    rows_v.at[p, pl.ds(t * 16, 16)],
                    out_hbm.at[pl.ds(seg0 + p * _HPT + t * 16, 16)])
                return carry

            lax.fori_loop(0, n16, wout, 0)

    for p in range(2):
        @pl.when(seg0 + p * _HPT + _HPT <= _NUM_SEGMENTS)
        def _drain_w():
            pltpu.make_async_copy(rows_v.at[p, pl.ds(0, _HPT)],
                                  out_hbm.at[pl.ds(seg0 + p * _HPT, _HPT)],
                                  sem_w.at[p]).wait()


@jax.jit
def _impl(enc_seq, segment_ids):
    ids3d = segment_ids.reshape(_NBLOCKS, _NSUB, _SUB)
    t_lo = jnp.sum((segment_ids < _SEG_HALF).astype(jnp.int32)).astype(jnp.int32)
    tlo16 = jnp.broadcast_to(t_lo, (16,))
    return _sc_mean(enc_seq, ids3d, tlo16)


def kernel(enc_seq, segment_ids):
    return _impl(enc_seq, segment_ids)


# speedup vs baseline: 1.0482x; 1.0038x over previous
import functools

import jax
import jax.numpy as jnp
from jax import lax
from jax.experimental import pallas as pl
from jax.experimental.pallas import tpu as pltpu
from jax.experimental.pallas import tpu_sc as plsc

_NUM_SEGMENTS = 10000
_SEG_HALF = 5120
_SEG_PAD = 2 * _SEG_HALF
_ACC_ROWS = _SEG_HALF + 8
_N_TOKENS = 320000
_D = 128
_SUB = 128
_BLOCK = 256
_NSUB = _BLOCK // _SUB
_NBLOCKS = _N_TOKENS // _BLOCK
_NC = 2
_NS = 16
_RPT = _SEG_HALF // _NS
_HPT = _RPT // 2


_mesh = plsc.VectorSubcoreMesh(core_axis_name="c", subcore_axis_name="s")


@functools.partial(
    pl.kernel,
    mesh=_mesh,
    out_type=jax.ShapeDtypeStruct((_NUM_SEGMENTS, _D), jnp.float32),
    scratch_types=[
        pltpu.VMEM((2, _NSUB, _SUB), jnp.int32),
        pltpu.VMEM((2, _NSUB, _SUB), jnp.int32),
        pltpu.VMEM((2, _BLOCK, _D), jnp.float32),
        pltpu.VMEM((_SUB,), jnp.float32),
        pltpu.VMEM((32, _D), jnp.float32),
        pltpu.VMEM((_RPT,), jnp.float32),
        pltpu.VMEM((16,), jnp.int32),
        pltpu.VMEM_SHARED((_ACC_ROWS, _D), jnp.float32),
        pltpu.VMEM_SHARED((_SEG_PAD,), jnp.float32),
        pltpu.SemaphoreType.DMA((2,)),
        pltpu.SemaphoreType.DMA((2,)),
        pltpu.SemaphoreType.DMA((2,)),
        pltpu.SemaphoreType.DMA((2,)),
    ],
)
def _sc_mean(enc_hbm, ids_hbm, tlo_hbm, out_hbm,
             idx_v, idx2_v, rows_v, ones_v, zero_v, cnt_v, tlo_v, acc_sh, cnt_sh,
             sem_rows, sem_ids, sem_w, sem_c):
    cid = lax.axis_index("c")
    sid = lax.axis_index("s")

    pltpu.sync_copy(tlo_hbm, tlo_v)

    for j in range(_SUB // 16):
        ones_v[pl.ds(j * 16, 16)] = jnp.ones((16,), jnp.float32)

    def zrow(r, carry):
        for j in range(_D // 16):
            zero_v[r, pl.ds(j * 16, 16)] = jnp.zeros((16,), jnp.float32)
        return carry

    lax.fori_loop(0, 32, zrow, 0)

    base_row = sid * _RPT

    t_lo = tlo_v[...][0]
    lo = jnp.where(cid == 0, 0, t_lo // _BLOCK)
    hi = jnp.where(cid == 0, (t_lo + _BLOCK - 1) // _BLOCK, _NBLOCKS)
    n_c = hi - lo
    per = n_c // _NS
    rem = n_c - per * _NS
    base = lo + sid * per + jnp.minimum(sid, rem)
    n_my = per + jnp.where(sid < rem, 1, 0)
    seg_base = cid * _SEG_HALF

    def _start_load(c, b):
        pltpu.async_copy(enc_hbm.at[pl.ds(c * _BLOCK, _BLOCK)], rows_v.at[b],
                         sem_rows.at[b])
        pltpu.async_copy(ids_hbm.at[c], idx_v.at[b], sem_ids.at[b])

    def _wait_load(c, b):
        pltpu.make_async_copy(enc_hbm.at[pl.ds(c * _BLOCK, _BLOCK)],
                              rows_v.at[b], sem_rows.at[b]).wait()
        pltpu.make_async_copy(ids_hbm.at[c], idx_v.at[b],
                              sem_ids.at[b]).wait()

    @pl.when(n_my > 0)
    def _prime():
        _start_load(base, 0)

    @pl.when(n_my > 1)
    def _prime2():
        _start_load(base + 1, 1)

    cnt_base = sid * (_SEG_PAD // _NS)

    def zacc(t, carry):
        pltpu.async_copy(zero_v, acc_sh.at[pl.ds(base_row + t * 32, 32)],
                         sem_w.at[0])
        return carry

    lax.fori_loop(0, _RPT // 32, zacc, 0)

    def zcnt(t, carry):
        pltpu.async_copy(zero_v.at[0], cnt_sh.at[pl.ds(cnt_base + t * 128, 128)],
                         sem_w.at[1])
        return carry

    lax.fori_loop(0, (_SEG_PAD // _NS) // 128, zcnt, 0)

    def zacc_d(t, carry):
        pltpu.make_async_copy(zero_v, acc_sh.at[pl.ds(base_row + t * 32, 32)],
                              sem_w.at[0]).wait()
        return carry

    lax.fori_loop(0, _RPT // 32, zacc_d, 0)

    def zcnt_d(t, carry):
        pltpu.make_async_copy(zero_v.at[0],
                              cnt_sh.at[pl.ds(cnt_base + t * 128, 128)],
                              sem_w.at[1]).wait()
        return carry

    lax.fori_loop(0, (_SEG_PAD // _NS) // 128, zcnt_d, 0)

    plsc.subcore_barrier()

    def body(i, carry):
        b = i % 2

        _wait_load(base + i, b)
        for j in range(_NSUB):
            pltpu.async_copy(ones_v, cnt_sh.at[idx_v.at[b, j]],
                             sem_c.at[b], add=True)
            for k in range(_SUB // 16):
                v = idx_v[b, j, pl.ds(k * 16, 16)] - seg_base
                oob = (v < 0) | (v >= _SEG_HALF)
                idx2_v[b, j, pl.ds(k * 16, 16)] = jnp.where(oob, _SEG_HALF, v)
            pltpu.sync_copy(rows_v.at[b, pl.ds(j * _SUB, _SUB)],
                            acc_sh.at[idx2_v.at[b, j]], add=True)
        for j in range(_NSUB):
            pltpu.make_async_copy(ones_v, cnt_sh.at[idx_v.at[b, j]],
                                  sem_c.at[b]).wait()

        @pl.when(i + 2 < n_my)
        def _next():
            _start_load(base + i + 2, b)

        return carry

    lax.fori_loop(0, n_my, body, 0)

    plsc.subcore_barrier()

    for p in range(2):
        pltpu.async_copy(acc_sh.at[pl.ds(base_row + p * _HPT, _HPT)],
                         rows_v.at[p, pl.ds(0, _HPT)], sem_rows.at[p])
    pltpu.sync_copy(cnt_sh.at[pl.ds(seg_base + base_row, _RPT)], cnt_v)

    def recip(k, carry):
        cv = cnt_v[pl.ds(k * 16, 16)]
        cnt_v[pl.ds(k * 16, 16)] = 1.0 / jnp.maximum(cv, 1.0)
        return carry

    lax.fori_loop(0, _RPT // 16, recip, 0)

    seg0 = seg_base + base_row
    for p in range(2):
        pltpu.make_async_copy(acc_sh.at[pl.ds(base_row + p * _HPT, _HPT)],
                              rows_v.at[p, pl.ds(0, _HPT)],
                              sem_rows.at[p]).wait()

        def divgrp(g, carry):
            m16 = cnt_v[pl.ds(p * _HPT + g * 16, 16)]
            for rr in range(16):
                r = g * 16 + rr
                m = lax.broadcast(m16[rr], (16,))
                for k in range(_D // 16):
                    rows_v[p, r, pl.ds(k * 16, 16)] = (
                        rows_v[p, r, pl.ds(k * 16, 16)] * m)
            return carry

        lax.fori_loop(0, _HPT // 16, divgrp, 0)
        fits = seg0 + p * _HPT + _HPT <= _NUM_SEGMENTS

        @pl.when(fits)
        def _bulk():
            pltpu.async_copy(rows_v.at[p, pl.ds(0, _HPT)],
                             out_hbm.at[pl.ds(seg0 + p * _HPT, _HPT)],
                             sem_w.at[p])

        n16 = jnp.clip(_NUM_SEGMENTS - (seg0 + p * _HPT), 0, _HPT) // 16

        @pl.when(jnp.logical_not(fits))
        def _tail():
            def wout(t, carry):
                pltpu.sync_copy(
                    rows_v.at[p, pl.ds(t * 16, 16)],
                    out_hbm.at[pl.ds(seg0 + p * _HPT + t * 16, 16)])
                return carry

            lax.fori_loop(0, n16, wout, 0)

    for p in range(2):
        @pl.when(seg0 + p * _HPT + _HPT <= _NUM_SEGMENTS)
        def _drain_w():
            pltpu.make_async_copy(rows_v.at[p, pl.ds(0, _HPT)],
                                  out_hbm.at[pl.ds(seg0 + p * _HPT, _HPT)],
                                  sem_w.at[p]).wait()


@jax.jit
def _impl(enc_seq, segment_ids):
    ids3d = segment_ids.reshape(_NBLOCKS, _NSUB, _SUB)
    t_lo = jnp.sum((segment_ids < _SEG_HALF).astype(jnp.int32)).astype(jnp.int32)
    tlo16 = jnp.broadcast_to(t_lo, (16,))
    return _sc_mean(enc_seq, ids3d, tlo16)


def kernel(enc_seq, segment_ids):
    return _impl(enc_seq, segment_ids)
